# baseline (speedup 1.0000x reference)
import jax
import jax.numpy as jnp
from jax import lax
from jax.experimental import pallas as pl
from jax.experimental.pallas import tpu as pltpu
from jax.experimental.pallas import tpu_sc as plsc

N_NODES = 10000
N_EDGES = 320000
D = 128
DH = D // 2
N_REL = 50

NC = 2
NS = 16
CHUNK = 128
CHUNKS_PER_W = 162
EPW = CHUNKS_PER_W * CHUNK
E_PAD = NS * EPW
N_PAD = 10016
ACC_ROWS = 10240
ROWS_PER_TILE = ACC_ROWS // NS
XT_ROWS_PER_TILE = N_PAD // NS


def _tc_prep(x_ref, wl_ref, wa_ref, rel_ref, xt_ref, s_ref, r_ref):
    xv = x_ref[...]
    xt = lax.dot_general(
        xv, wl_ref[...], (((1,), (1,)), ((), ())),
        preferred_element_type=jnp.float32)
    pad_z = jnp.zeros((N_PAD - N_NODES, DH), jnp.float32)
    xt_ref[0, 0:N_NODES, :] = xt[:, 0:DH]
    xt_ref[0, N_NODES:N_PAD, :] = pad_z
    xt_ref[1, 0:N_NODES, :] = xt[:, DH:D]
    xt_ref[1, N_NODES:N_PAD, :] = pad_z
    wa = wa_ref[...]
    sv = lax.dot_general(
        xv, wa, (((1,), (1,)), ((), ())), preferred_element_type=jnp.float32)
    s_ref[0:N_NODES, :] = sv
    s_ref[N_NODES:N_PAD, :] = jnp.zeros((N_PAD - N_NODES, 1), jnp.float32)
    r_ref[...] = lax.dot_general(
        rel_ref[...], wa, (((1,), (1,)), ((), ())),
        preferred_element_type=jnp.float32)


def _sc_edges(xt_hbm, s_hbm, r_hbm, src_hbm, tgt_hbm, typ_hbm, out_hbm,
              s_v, r_v, srcb, tgtb, typb,
              rows0, rows1, rows2, xts, acc,
              gsem, gsemb, ssem, tsem, tgsem, srcsem):
    c = lax.axis_index("c")
    s = lax.axis_index("s")
    bufs = (rows0, rows1, rows2)
    xt_half = xt_hbm.at[c]

    @pl.loop(0, CHUNK)
    def _zero_rows(i):
        zero16 = jnp.zeros((16,), jnp.float32)
        for h in range(DH // 16):
            rows0[i, pl.ds(h * 16, 16)] = zero16

    for b in range(ROWS_PER_TILE // CHUNK):
        pltpu.sync_copy(rows0, acc.at[pl.ds((s * 5 + b) * CHUNK, CHUNK)])

    pltpu.sync_copy(
        xt_hbm.at[c, pl.ds(s * XT_ROWS_PER_TILE, XT_ROWS_PER_TILE)],
        xts.at[pl.ds(s * XT_ROWS_PER_TILE, XT_ROWS_PER_TILE)])
    pltpu.sync_copy(s_hbm, s_v)
    pltpu.sync_copy(r_hbm, r_v)
    plsc.subcore_barrier()

    def _scale(r6, rows_x, X):
        @pl.loop(0, CHUNK // 16)
        def _grp(k):
            sl = pl.ds(k * 16, 16)
            idx16 = srcb[r6, sl]
            typ16 = typb[X, sl]
            sv = plsc.load_gather(s_v, [idx16])
            rv = plsc.load_gather(r_v, [typ16])
            a16 = 1.0 / (1.0 + jnp.exp(-(sv + rv)))
            base = k * 16
            for l in range(16):
                a = lax.broadcast_in_dim(a16[l], (16,), ())
                for h in range(DH // 16):
                    fsl = pl.ds(h * 16, 16)
                    rows_x[base + l, fsl] = rows_x[base + l, fsl] * a

    HB = CHUNK // 2

    def _src_pre(j, r6):
        pltpu.async_copy(src_hbm.at[s, j], srcb.at[r6], srcsem.at[r6])

    def _tgt_pre(j, r3):
        pltpu.async_copy(tgt_hbm.at[s, j], tgtb.at[r3], tgsem.at[r3])

    def _typ_pre(j, r3):
        pltpu.async_copy(typ_hbm.at[s, j], typb.at[r3], tsem.at[r3])

    def _gth(j, r6, r3):
        pltpu.make_async_copy(
            src_hbm.at[s, j], srcb.at[r6], srcsem.at[r6]).wait()
        if r3 == 2:
            pltpu.async_copy(xts.at[srcb.at[r6]], bufs[r3], gsem.at[r3])
        else:
            pltpu.async_copy(xt_half.at[srcb.at[r6, pl.ds(0, HB)]],
                             bufs[r3].at[pl.ds(0, HB)], gsem.at[r3])
            pltpu.async_copy(xt_half.at[srcb.at[r6, pl.ds(HB, HB)]],
                             bufs[r3].at[pl.ds(HB, HB)], gsemb.at[r3])

    def _gth_wait(j, r6, r3):
        if r3 == 2:
            pltpu.make_async_copy(
                xts.at[srcb.at[r6]], bufs[r3], gsem.at[r3]).wait()
        else:
            pltpu.make_async_copy(
                xt_half.at[srcb.at[r6, pl.ds(0, HB)]],
                bufs[r3].at[pl.ds(0, HB)], gsem.at[r3]).wait()
            pltpu.make_async_copy(
                xt_half.at[srcb.at[r6, pl.ds(HB, HB)]],
                bufs[r3].at[pl.ds(HB, HB)], gsemb.at[r3]).wait()

    for j in range(4):
        _src_pre(j, j % 6)
    for j in range(2):
        _tgt_pre(j, j % 3)
        _typ_pre(j, j % 3)
        _gth(j, j % 6, j % 3)

    @pl.loop(0, CHUNKS_PER_W, step=6)
    def _t(t):
        for i in range(6):
            j = t + i
            X = i % 3
            Z = (i + 2) % 3
            _gth_wait(j, i % 6, X)
            pltpu.make_async_copy(
                typ_hbm.at[s, j], typb.at[X], tsem.at[X]).wait()
            _scale(i % 6, bufs[X], X)
            if i == 0:
                @pl.when(t >= 1)
                def _():
                    pltpu.make_async_copy(
                        bufs[Z], acc.at[tgtb.at[Z]], ssem.at[Z]).wait()
            else:
                pltpu.make_async_copy(
                    bufs[Z], acc.at[tgtb.at[Z]], ssem.at[Z]).wait()
            if i >= 4:
                @pl.when(j + 2 < CHUNKS_PER_W)
                def _():
                    _gth(j + 2, (i + 2) % 6, Z)
                    _typ_pre(j + 2, Z)
                    _tgt_pre(j + 2, Z)
            else:
                _gth(j + 2, (i + 2) % 6, Z)
                _typ_pre(j + 2, Z)
                _tgt_pre(j + 2, Z)
            pltpu.make_async_copy(
                tgt_hbm.at[s, j], tgtb.at[X], tgsem.at[X]).wait()
            pltpu.async_copy(bufs[X], acc.at[tgtb.at[X]], ssem.at[X],
                             add=True)
            if i >= 2:
                @pl.when(j + 4 < CHUNKS_PER_W)
                def _():
                    _src_pre(j + 4, (i + 4) % 6)
            else:
                _src_pre(j + 4, (i + 4) % 6)

    pltpu.make_async_copy(bufs[2], acc.at[tgtb.at[2]], ssem.at[2]).wait()

    plsc.subcore_barrier()
    for b in range(5):
        rbase = s * 625 + b * 125
        pltpu.sync_copy(acc.at[pl.ds(rbase, 125)], rows0.at[pl.ds(0, 125)])

        @pl.loop(0, 125)
        def _relu(i):
            for h in range(DH // 16):
                fsl = pl.ds(h * 16, 16)
                rows0[i, fsl] = jnp.maximum(rows0[i, fsl], 0.0)

        pltpu.sync_copy(rows0.at[pl.ds(0, 125)],
                        out_hbm.at[pl.ds(rbase, 125), pl.ds(c * DH, DH)])


@jax.jit
def _run(x, edge_index, edge_type, rel_emb, W_lin, W_attn):
    src = edge_index[0].astype(jnp.int32)
    tgt = edge_index[1].astype(jnp.int32)
    typ = edge_type.astype(jnp.int32)

    pad = E_PAD - N_EDGES
    src = jnp.concatenate([src, jnp.full((pad,), N_NODES, jnp.int32)])
    tgt = jnp.concatenate([tgt, jnp.zeros((pad,), jnp.int32)])
    typ = jnp.concatenate([typ, jnp.zeros((pad,), jnp.int32)])
    src = src.reshape(NS, CHUNKS_PER_W, CHUNK)
    tgt = tgt.reshape(NS, CHUNKS_PER_W, CHUNK)
    typ = typ.reshape(NS, CHUNKS_PER_W, CHUNK)

    rel_pad = jnp.concatenate(
        [rel_emb, jnp.zeros((64 - N_REL, D), jnp.float32)], axis=0)

    xt3, s_pad, r_pad = pl.pallas_call(
        _tc_prep,
        out_shape=[
            jax.ShapeDtypeStruct((NC, N_PAD, DH), jnp.float32),
            jax.ShapeDtypeStruct((N_PAD, 1), jnp.float32),
            jax.ShapeDtypeStruct((64, 1), jnp.float32),
        ],
    )(x, W_lin, W_attn, rel_pad)

    s1 = s_pad.reshape(N_PAD)
    r1 = r_pad.reshape(64)

    mesh = plsc.VectorSubcoreMesh(
        core_axis_name="c", subcore_axis_name="s",
        num_cores=NC, num_subcores=NS)
    sc_call = pl.kernel(
        _sc_edges,
        out_type=jax.ShapeDtypeStruct((N_NODES, D), jnp.float32),
        mesh=mesh,
        compiler_params=pltpu.CompilerParams(
            needs_layout_passes=False, use_tc_tiling_on_sc=False),
        scratch_types=[
            pltpu.VMEM((N_PAD,), jnp.float32),
            pltpu.VMEM((64,), jnp.float32),
            pltpu.VMEM((6, CHUNK), jnp.int32),
            pltpu.VMEM((3, CHUNK), jnp.int32),
            pltpu.VMEM((3, CHUNK), jnp.int32),
            pltpu.VMEM((CHUNK, DH), jnp.float32),
            pltpu.VMEM((CHUNK, DH), jnp.float32),
            pltpu.VMEM((CHUNK, DH), jnp.float32),
            pltpu.VMEM_SHARED((N_PAD, DH), jnp.float32),
            pltpu.VMEM_SHARED((ACC_ROWS, DH), jnp.float32),
            pltpu.SemaphoreType.DMA((3,)),
            pltpu.SemaphoreType.DMA((3,)),
            pltpu.SemaphoreType.DMA((3,)),
            pltpu.SemaphoreType.DMA((3,)),
            pltpu.SemaphoreType.DMA((3,)),
            pltpu.SemaphoreType.DMA((6,)),
        ],
    )
    out = sc_call(xt3, s1, r1, src, tgt, typ)
    return out


def kernel(x, edge_index, edge_type, rel_emb, W_lin, W_attn):
    return _run(x, edge_index, edge_type, rel_emb, W_lin, W_attn)


# speedup vs baseline: 1.4209x; 1.4209x over previous
import jax
import jax.numpy as jnp
from jax import lax
from jax.experimental import pallas as pl
from jax.experimental.pallas import tpu as pltpu
from jax.experimental.pallas import tpu_sc as plsc

N_NODES = 10000
N_EDGES = 320000
D = 128
DH = D // 2
N_REL = 50

NC = 2
NS = 16
CHUNK = 128
CHUNKS_PER_W = 160
EPW = CHUNKS_PER_W * CHUNK
E_PAD = NS * EPW
N_PAD = 10016
ACC_ROWS = 10240
ROWS_PER_TILE = ACC_ROWS // NS


def _tc_prep(x_ref, wl_ref, wa_ref, rel_ref, xt_ref, s_ref, r_ref):
    xv = x_ref[...]
    xt = lax.dot_general(
        xv, wl_ref[...], (((1,), (1,)), ((), ())),
        preferred_element_type=jnp.float32)
    pad_z = jnp.zeros((N_PAD - N_NODES, DH), jnp.float32)
    xt_ref[0:N_NODES, :] = xt[:, 0:DH]
    xt_ref[N_NODES:N_PAD, :] = pad_z
    xt_ref[N_PAD:N_PAD + N_NODES, :] = xt[:, DH:D]
    xt_ref[N_PAD + N_NODES:2 * N_PAD, :] = pad_z
    wa = wa_ref[...]
    sv = lax.dot_general(
        xv, wa, (((1,), (1,)), ((), ())), preferred_element_type=jnp.float32)
    s_ref[0:N_NODES, :] = sv
    s_ref[N_NODES:N_PAD, :] = jnp.zeros((N_PAD - N_NODES, 1), jnp.float32)
    r_ref[...] = lax.dot_general(
        rel_ref[...], wa, (((1,), (1,)), ((), ())),
        preferred_element_type=jnp.float32)


def _sc_edges(xt_hbm, s_hbm, r_hbm, src_hbm, tgt_hbm, typ_hbm, out_hbm,
              src_v, tgt_v, s_v, r_v, typb,
              rows0, rows1, rows2, rows3, acc,
              gsem, gsemb, ssem, tsem):
    c = lax.axis_index("c")
    s = lax.axis_index("s")
    bufs = (rows0, rows1, rows2, rows3)

    @pl.loop(0, CHUNK)
    def _zero_rows(i):
        zero16 = jnp.zeros((16,), jnp.float32)
        for h in range(DH // 16):
            rows0[i, pl.ds(h * 16, 16)] = zero16

    for b in range(ROWS_PER_TILE // CHUNK):
        pltpu.sync_copy(rows0, acc.at[pl.ds((s * 5 + b) * CHUNK, CHUNK)])
    plsc.subcore_barrier()

    pltpu.sync_copy(src_hbm.at[c, s], src_v)
    pltpu.sync_copy(tgt_hbm.at[s], tgt_v)
    pltpu.sync_copy(s_hbm, s_v)
    pltpu.sync_copy(r_hbm, r_v)

    coff = c * N_PAD

    def _scale(j, rows_x, X):
        @pl.loop(0, CHUNK // 16)
        def _grp(k):
            sl = pl.ds(k * 16, 16)
            idx16 = src_v[j, sl] - coff
            typ16 = typb[X, sl]
            sv = plsc.load_gather(s_v, [idx16])
            rv = plsc.load_gather(r_v, [typ16])
            a16 = 1.0 / (1.0 + jnp.exp(-(sv + rv)))
            base = k * 16
            for l in range(16):
                a = lax.broadcast_in_dim(a16[l], (16,), ())
                for h in range(DH // 16):
                    fsl = pl.ds(h * 16, 16)
                    rows_x[base + l, fsl] = rows_x[base + l, fsl] * a

    HB = CHUNK // 2

    def _gth(j, X):
        pltpu.async_copy(xt_hbm.at[src_v.at[j, pl.ds(0, HB)]],
                         bufs[X].at[pl.ds(0, HB)], gsem.at[X])
        pltpu.async_copy(xt_hbm.at[src_v.at[j, pl.ds(HB, HB)]],
                         bufs[X].at[pl.ds(HB, HB)], gsemb.at[X])

    def _gth_wait(j, X):
        pltpu.make_async_copy(xt_hbm.at[src_v.at[j, pl.ds(0, HB)]],
                              bufs[X].at[pl.ds(0, HB)], gsem.at[X]).wait()
        pltpu.make_async_copy(xt_hbm.at[src_v.at[j, pl.ds(HB, HB)]],
                              bufs[X].at[pl.ds(HB, HB)], gsemb.at[X]).wait()

    def _typ_pre(j, X):
        pltpu.async_copy(typ_hbm.at[s, j], typb.at[X], tsem.at[X])

    for j in range(2):
        _gth(j, j)
        _typ_pre(j, j)

    @pl.loop(0, CHUNKS_PER_W, step=4)
    def _t(t):
        for i in range(4):
            j = t + i
            X = i
            Z = (i + 2) % 4
            W = (i + 3) % 4
            _gth_wait(j, X)
            pltpu.make_async_copy(
                typ_hbm.at[s, j], typb.at[X], tsem.at[X]).wait()
            if i < 2:
                _gth(j + 2, Z)
                _typ_pre(j + 2, Z)
            else:
                @pl.when(j + 2 < CHUNKS_PER_W)
                def _():
                    _gth(j + 2, Z)
                    _typ_pre(j + 2, Z)
            _scale(j, bufs[X], X)
            if i == 0:
                @pl.when(t >= 1)
                def _():
                    pltpu.make_async_copy(
                        bufs[W], acc.at[tgt_v.at[j - 1]], ssem.at[W]).wait()
            else:
                pltpu.make_async_copy(
                    bufs[W], acc.at[tgt_v.at[j - 1]], ssem.at[W]).wait()
            pltpu.async_copy(bufs[X], acc.at[tgt_v.at[j]], ssem.at[X],
                             add=True)

    pltpu.make_async_copy(
        bufs[3], acc.at[tgt_v.at[CHUNKS_PER_W - 1]], ssem.at[3]).wait()

    plsc.subcore_barrier()
    for b in range(5):
        rbase = s * 625 + b * 125
        pltpu.sync_copy(acc.at[pl.ds(rbase, 125)], rows0.at[pl.ds(0, 125)])

        @pl.loop(0, 125)
        def _relu(i):
            for h in range(DH // 16):
                fsl = pl.ds(h * 16, 16)
                rows0[i, fsl] = jnp.maximum(rows0[i, fsl], 0.0)

        pltpu.sync_copy(rows0.at[pl.ds(0, 125)],
                        out_hbm.at[pl.ds(rbase, 125), pl.ds(c * DH, DH)])


@jax.jit
def _run(x, edge_index, edge_type, rel_emb, W_lin, W_attn):
    src = edge_index[0].astype(jnp.int32)
    tgt = edge_index[1].astype(jnp.int32)
    typ = edge_type.astype(jnp.int32)

    pad = E_PAD - N_EDGES
    src = jnp.concatenate([src, jnp.full((pad,), N_NODES, jnp.int32)])
    tgt = jnp.concatenate([tgt, jnp.zeros((pad,), jnp.int32)])
    typ = jnp.concatenate([typ, jnp.zeros((pad,), jnp.int32)])
    src = src.reshape(NS, CHUNKS_PER_W, CHUNK)
    tgt = tgt.reshape(NS, CHUNKS_PER_W, CHUNK)
    typ = typ.reshape(NS, CHUNKS_PER_W, CHUNK)
    src_off = src[None] + (jnp.arange(NC, dtype=jnp.int32) * N_PAD)[
        :, None, None, None]

    rel_pad = jnp.concatenate(
        [rel_emb, jnp.zeros((64 - N_REL, D), jnp.float32)], axis=0)

    xt_split, s_pad, r_pad = pl.pallas_call(
        _tc_prep,
        out_shape=[
            jax.ShapeDtypeStruct((NC * N_PAD, DH), jnp.float32),
            jax.ShapeDtypeStruct((N_PAD, 1), jnp.float32),
            jax.ShapeDtypeStruct((64, 1), jnp.float32),
        ],
    )(x, W_lin, W_attn, rel_pad)

    s1 = s_pad.reshape(N_PAD)
    r1 = r_pad.reshape(64)

    mesh = plsc.VectorSubcoreMesh(
        core_axis_name="c", subcore_axis_name="s",
        num_cores=NC, num_subcores=NS)
    sc_call = pl.kernel(
        _sc_edges,
        out_type=jax.ShapeDtypeStruct((N_NODES, D), jnp.float32),
        mesh=mesh,
        compiler_params=pltpu.CompilerParams(
            needs_layout_passes=False, use_tc_tiling_on_sc=False),
        scratch_types=[
            pltpu.VMEM((CHUNKS_PER_W, CHUNK), jnp.int32),
            pltpu.VMEM((CHUNKS_PER_W, CHUNK), jnp.int32),
            pltpu.VMEM((N_PAD,), jnp.float32),
            pltpu.VMEM((64,), jnp.float32),
            pltpu.VMEM((4, CHUNK), jnp.int32),
            pltpu.VMEM((CHUNK, DH), jnp.float32),
            pltpu.VMEM((CHUNK, DH), jnp.float32),
            pltpu.VMEM((CHUNK, DH), jnp.float32),
            pltpu.VMEM((CHUNK, DH), jnp.float32),
            pltpu.VMEM_SHARED((ACC_ROWS, DH), jnp.float32),
            pltpu.SemaphoreType.DMA((4,)),
            pltpu.SemaphoreType.DMA((4,)),
            pltpu.SemaphoreType.DMA((4,)),
            pltpu.SemaphoreType.DMA((4,)),
        ],
    )
    out = sc_call(xt_split, s1, r1, src_off, tgt, typ)
    return out


def kernel(x, edge_index, edge_type, rel_emb, W_lin, W_attn):
    return _run(x, edge_index, edge_type, rel_emb, W_lin, W_attn)


# speedup vs baseline: 1.6066x; 1.1307x over previous
import jax
import jax.numpy as jnp
from jax import lax
from jax.experimental import pallas as pl
from jax.experimental.pallas import tpu as pltpu
from jax.experimental.pallas import tpu_sc as plsc

N_NODES = 10000
N_EDGES = 320000
D = 128
DH = D // 2
N_REL = 50

NC = 2
NS = 16
CHUNK = 128
CHUNKS_PER_W = 159
EPW = CHUNKS_PER_W * CHUNK
E_PAD = NS * EPW
N_PAD = 10016
ACC_ROWS = 10240
ROWS_PER_TILE = ACC_ROWS // NS


def _tc_prep(x_ref, wl_ref, wa_ref, rel_ref, xt_ref, s_ref, r_ref):
    xv = x_ref[...]
    xt = lax.dot_general(
        xv, wl_ref[...], (((1,), (1,)), ((), ())),
        preferred_element_type=jnp.float32)
    pad_z = jnp.zeros((N_PAD - N_NODES, DH), jnp.float32)
    xt_ref[0:N_NODES, :] = xt[:, 0:DH]
    xt_ref[N_NODES:N_PAD, :] = pad_z
    xt_ref[N_PAD:N_PAD + N_NODES, :] = xt[:, DH:D]
    xt_ref[N_PAD + N_NODES:2 * N_PAD, :] = pad_z
    wa = wa_ref[...]
    sv = lax.dot_general(
        xv, wa, (((1,), (1,)), ((), ())), preferred_element_type=jnp.float32)
    s_ref[0:N_NODES, :] = sv
    s_ref[N_NODES:N_PAD, :] = jnp.zeros((N_PAD - N_NODES, 1), jnp.float32)
    r_ref[...] = lax.dot_general(
        rel_ref[...], wa, (((1,), (1,)), ((), ())),
        preferred_element_type=jnp.float32)


def _sc_edges(xt_hbm, s_hbm, r_hbm, src_hbm, tgt_hbm, typ_hbm, out_hbm,
              src_v, tgt_v, s_v, r_v, typb,
              rows0, rows1, rows2, acc,
              gsem0, gsem1, gsem2, ssem0, ssem1, ssem2,
              tsem0, tsem1, tsem2, gsem0b, gsem1b, gsem2b):
    c = lax.axis_index("c")
    s = lax.axis_index("s")
    bufs = (rows0, rows1, rows2)
    gsems = (gsem0, gsem1, gsem2)
    gsembs = (gsem0b, gsem1b, gsem2b)
    ssems = (ssem0, ssem1, ssem2)
    tsems = (tsem0, tsem1, tsem2)

    @pl.loop(0, CHUNK)
    def _zero_rows(i):
        zero16 = jnp.zeros((16,), jnp.float32)
        for h in range(DH // 16):
            rows0[i, pl.ds(h * 16, 16)] = zero16

    for b in range(ROWS_PER_TILE // CHUNK):
        pltpu.sync_copy(rows0, acc.at[pl.ds((s * 5 + b) * CHUNK, CHUNK)])
    plsc.subcore_barrier()

    pltpu.sync_copy(src_hbm.at[c, s], src_v)
    pltpu.sync_copy(tgt_hbm.at[s], tgt_v)
    pltpu.sync_copy(s_hbm, s_v)
    pltpu.sync_copy(r_hbm, r_v)

    coff = c * N_PAD

    def _scale(j, rows_x, X):
        @pl.loop(0, CHUNK // 16)
        def _grp(k):
            sl = pl.ds(k * 16, 16)
            idx16 = src_v[j, sl] - coff
            typ16 = typb[X, sl]
            sv = plsc.load_gather(s_v, [idx16])
            rv = plsc.load_gather(r_v, [typ16])
            a16 = 1.0 / (1.0 + jnp.exp(-(sv + rv)))
            base = k * 16
            for l in range(16):
                a = lax.broadcast_in_dim(a16[l], (16,), ())
                for h in range(DH // 16):
                    fsl = pl.ds(h * 16, 16)
                    rows_x[base + l, fsl] = rows_x[base + l, fsl] * a

    HB = CHUNK // 2

    def _gth(j, X):
        pltpu.async_copy(xt_hbm.at[src_v.at[j, pl.ds(0, HB)]],
                         bufs[X].at[pl.ds(0, HB)], gsems[X])
        pltpu.async_copy(xt_hbm.at[src_v.at[j, pl.ds(HB, HB)]],
                         bufs[X].at[pl.ds(HB, HB)], gsembs[X])

    def _gth_wait(j, X):
        pltpu.make_async_copy(xt_hbm.at[src_v.at[j, pl.ds(0, HB)]],
                              bufs[X].at[pl.ds(0, HB)], gsems[X]).wait()
        pltpu.make_async_copy(xt_hbm.at[src_v.at[j, pl.ds(HB, HB)]],
                              bufs[X].at[pl.ds(HB, HB)], gsembs[X]).wait()

    _gth(0, 0)
    _gth(1, 1)
    pltpu.async_copy(typ_hbm.at[s, 0], typb.at[0], tsem0)
    pltpu.async_copy(typ_hbm.at[s, 1], typb.at[1], tsem1)

    @pl.loop(0, CHUNKS_PER_W, step=3)
    def _t(t):
        for i in range(3):
            j = t + i
            X = i
            Z = (i + 2) % 3
            _gth_wait(j, X)
            pltpu.make_async_copy(
                typ_hbm.at[s, j], typb.at[X], tsems[X]).wait()
            _scale(j, bufs[X], X)
            if i == 0:
                @pl.when(t >= 1)
                def _():
                    pltpu.make_async_copy(
                        bufs[Z], acc.at[tgt_v.at[j - 1]], ssems[Z]).wait()
                _gth(j + 2, Z)
                pltpu.async_copy(typ_hbm.at[s, j + 2], typb.at[Z], tsems[Z])
            else:
                pltpu.make_async_copy(
                    bufs[Z], acc.at[tgt_v.at[j - 1]], ssems[Z]).wait()

                @pl.when(j + 2 < CHUNKS_PER_W)
                def _():
                    _gth(j + 2, Z)
                    pltpu.async_copy(
                        typ_hbm.at[s, j + 2], typb.at[Z], tsems[Z])
            pltpu.async_copy(bufs[X], acc.at[tgt_v.at[j]], ssems[X], add=True)

    pltpu.make_async_copy(
        bufs[2], acc.at[tgt_v.at[CHUNKS_PER_W - 1]], ssems[2]).wait()

    plsc.subcore_barrier()
    for b in range(5):
        rbase = s * 625 + b * 125
        pltpu.sync_copy(acc.at[pl.ds(rbase, 125)], rows0.at[pl.ds(0, 125)])

        @pl.loop(0, 125)
        def _relu(i):
            for h in range(DH // 16):
                fsl = pl.ds(h * 16, 16)
                rows0[i, fsl] = jnp.maximum(rows0[i, fsl], 0.0)

        pltpu.sync_copy(rows0.at[pl.ds(0, 125)],
                        out_hbm.at[pl.ds(rbase, 125), pl.ds(c * DH, DH)])


@jax.jit
def _run(x, edge_index, edge_type, rel_emb, W_lin, W_attn):
    src = edge_index[0].astype(jnp.int32)
    tgt = edge_index[1].astype(jnp.int32)
    typ = edge_type.astype(jnp.int32)

    pad = E_PAD - N_EDGES
    src = jnp.concatenate([src, jnp.full((pad,), N_NODES, jnp.int32)])
    tgt = jnp.concatenate([tgt, jnp.zeros((pad,), jnp.int32)])
    typ = jnp.concatenate([typ, jnp.zeros((pad,), jnp.int32)])
    src = src.reshape(NS, CHUNKS_PER_W, CHUNK)
    tgt = tgt.reshape(NS, CHUNKS_PER_W, CHUNK)
    typ = typ.reshape(NS, CHUNKS_PER_W, CHUNK)
    src_off = src[None] + (jnp.arange(NC, dtype=jnp.int32) * N_PAD)[
        :, None, None, None]

    rel_pad = jnp.concatenate(
        [rel_emb, jnp.zeros((64 - N_REL, D), jnp.float32)], axis=0)

    xt_split, s_pad, r_pad = pl.pallas_call(
        _tc_prep,
        out_shape=[
            jax.ShapeDtypeStruct((NC * N_PAD, DH), jnp.float32),
            jax.ShapeDtypeStruct((N_PAD, 1), jnp.float32),
            jax.ShapeDtypeStruct((64, 1), jnp.float32),
        ],
    )(x, W_lin, W_attn, rel_pad)

    s1 = s_pad.reshape(N_PAD)
    r1 = r_pad.reshape(64)

    mesh = plsc.VectorSubcoreMesh(
        core_axis_name="c", subcore_axis_name="s",
        num_cores=NC, num_subcores=NS)
    sc_call = pl.kernel(
        _sc_edges,
        out_type=jax.ShapeDtypeStruct((N_NODES, D), jnp.float32),
        mesh=mesh,
        compiler_params=pltpu.CompilerParams(
            needs_layout_passes=False, use_tc_tiling_on_sc=False),
        scratch_types=[
            pltpu.VMEM((CHUNKS_PER_W, CHUNK), jnp.int32),
            pltpu.VMEM((CHUNKS_PER_W, CHUNK), jnp.int32),
            pltpu.VMEM((N_PAD,), jnp.float32),
            pltpu.VMEM((64,), jnp.float32),
            pltpu.VMEM((3, CHUNK), jnp.int32),
            pltpu.VMEM((CHUNK, DH), jnp.float32),
            pltpu.VMEM((CHUNK, DH), jnp.float32),
            pltpu.VMEM((CHUNK, DH), jnp.float32),
            pltpu.VMEM_SHARED((ACC_ROWS, DH), jnp.float32),
            pltpu.SemaphoreType.DMA,
            pltpu.SemaphoreType.DMA,
            pltpu.SemaphoreType.DMA,
            pltpu.SemaphoreType.DMA,
            pltpu.SemaphoreType.DMA,
            pltpu.SemaphoreType.DMA,
            pltpu.SemaphoreType.DMA,
            pltpu.SemaphoreType.DMA,
            pltpu.SemaphoreType.DMA,
            pltpu.SemaphoreType.DMA,
            pltpu.SemaphoreType.DMA,
            pltpu.SemaphoreType.DMA,
        ],
    )
    out = sc_call(xt_split, s1, r1, src_off, tgt, typ)
    return out


def kernel(x, edge_index, edge_type, rel_emb, W_lin, W_attn):
    return _run(x, edge_index, edge_type, rel_emb, W_lin, W_attn)


# speedup vs baseline: 1.6763x; 1.0434x over previous
import jax
import jax.numpy as jnp
from jax import lax
from jax.experimental import pallas as pl
from jax.experimental.pallas import tpu as pltpu
from jax.experimental.pallas import tpu_sc as plsc

N_NODES = 10000
N_EDGES = 320000
D = 128
DH = D // 2
N_REL = 50

NC = 2
NS = 16
CHUNK = 128
CHUNKS_PER_W = 159
EPW = CHUNKS_PER_W * CHUNK
E_PAD = NS * EPW
N_PAD = 10016
ACC_ROWS = 10240
ROWS_PER_TILE = ACC_ROWS // NS


def _tc_prep(x_ref, wl_ref, wa_ref, rel_ref, xt_ref, s_ref, r_ref):
    xv = x_ref[...]
    xt = lax.dot_general(
        xv, wl_ref[...], (((1,), (1,)), ((), ())),
        preferred_element_type=jnp.float32)
    pad_z = jnp.zeros((N_PAD - N_NODES, DH), jnp.float32)
    xt_ref[0:N_NODES, :] = xt[:, 0:DH]
    xt_ref[N_NODES:N_PAD, :] = pad_z
    xt_ref[N_PAD:N_PAD + N_NODES, :] = xt[:, DH:D]
    xt_ref[N_PAD + N_NODES:2 * N_PAD, :] = pad_z
    wa = wa_ref[...]
    sv = lax.dot_general(
        xv, wa, (((1,), (1,)), ((), ())), preferred_element_type=jnp.float32)
    s_ref[0:N_NODES, :] = sv
    s_ref[N_NODES:N_PAD, :] = jnp.zeros((N_PAD - N_NODES, 1), jnp.float32)
    r_ref[...] = lax.dot_general(
        rel_ref[...], wa, (((1,), (1,)), ((), ())),
        preferred_element_type=jnp.float32)


def _sc_edges(xt_hbm, s_hbm, r_hbm, src_hbm, tgt_hbm, typ_hbm, out_hbm,
              src_v, tgt_v, s_v, r_v, typb,
              rows0, rows1, rows2, acc,
              gsem0, gsem1, gsem2, ssem0, ssem1, ssem2,
              tsem0, tsem1, tsem2, gsem0b, gsem1b, gsem2b):
    c = lax.axis_index("c")
    s = lax.axis_index("s")
    bufs = (rows0, rows1, rows2)
    gsems = (gsem0, gsem1, gsem2)
    gsembs = (gsem0b, gsem1b, gsem2b)
    ssems = (ssem0, ssem1, ssem2)
    tsems = (tsem0, tsem1, tsem2)

    @pl.loop(0, CHUNK)
    def _zero_rows(i):
        zero16 = jnp.zeros((16,), jnp.float32)
        for h in range(DH // 16):
            rows0[i, pl.ds(h * 16, 16)] = zero16

    for b in range(ROWS_PER_TILE // CHUNK):
        pltpu.sync_copy(rows0, acc.at[pl.ds((s * 5 + b) * CHUNK, CHUNK)])
    plsc.subcore_barrier()

    pltpu.sync_copy(src_hbm.at[c, s], src_v)
    pltpu.sync_copy(tgt_hbm.at[s], tgt_v)
    pltpu.sync_copy(s_hbm, s_v)
    pltpu.sync_copy(r_hbm, r_v)

    coff = c * N_PAD

    def _scale(j, rows_x, X):
        @plsc.parallel_loop(0, CHUNK // 16, unroll=2)
        def _grp(k):
            sl = pl.ds(k * 16, 16)
            idx16 = src_v[j, sl] - coff
            typ16 = typb[X, sl]
            sv = plsc.load_gather(s_v, [idx16])
            rv = plsc.load_gather(r_v, [typ16])
            a16 = 1.0 / (1.0 + jnp.exp(-(sv + rv)))
            base = k * 16
            for l in range(16):
                a = lax.broadcast_in_dim(a16[l], (16,), ())
                for h in range(DH // 16):
                    fsl = pl.ds(h * 16, 16)
                    rows_x[base + l, fsl] = rows_x[base + l, fsl] * a

    HB = CHUNK // 2

    def _gth(j, X):
        pltpu.async_copy(xt_hbm.at[src_v.at[j, pl.ds(0, HB)]],
                         bufs[X].at[pl.ds(0, HB)], gsems[X])
        pltpu.async_copy(xt_hbm.at[src_v.at[j, pl.ds(HB, HB)]],
                         bufs[X].at[pl.ds(HB, HB)], gsembs[X])

    def _gth_wait(j, X):
        pltpu.make_async_copy(xt_hbm.at[src_v.at[j, pl.ds(0, HB)]],
                              bufs[X].at[pl.ds(0, HB)], gsems[X]).wait()
        pltpu.make_async_copy(xt_hbm.at[src_v.at[j, pl.ds(HB, HB)]],
                              bufs[X].at[pl.ds(HB, HB)], gsembs[X]).wait()

    _gth(0, 0)
    _gth(1, 1)
    pltpu.async_copy(typ_hbm.at[s, 0], typb.at[0], tsem0)
    pltpu.async_copy(typ_hbm.at[s, 1], typb.at[1], tsem1)

    @pl.loop(0, CHUNKS_PER_W, step=3)
    def _t(t):
        for i in range(3):
            j = t + i
            X = i
            Z = (i + 2) % 3
            _gth_wait(j, X)
            pltpu.make_async_copy(
                typ_hbm.at[s, j], typb.at[X], tsems[X]).wait()
            _scale(j, bufs[X], X)
            if i == 0:
                @pl.when(t >= 1)
                def _():
                    pltpu.make_async_copy(
                        bufs[Z], acc.at[tgt_v.at[j - 1]], ssems[Z]).wait()
                _gth(j + 2, Z)
                pltpu.async_copy(typ_hbm.at[s, j + 2], typb.at[Z], tsems[Z])
            else:
                pltpu.make_async_copy(
                    bufs[Z], acc.at[tgt_v.at[j - 1]], ssems[Z]).wait()

                @pl.when(j + 2 < CHUNKS_PER_W)
                def _():
                    _gth(j + 2, Z)
                    pltpu.async_copy(
                        typ_hbm.at[s, j + 2], typb.at[Z], tsems[Z])
            pltpu.async_copy(bufs[X], acc.at[tgt_v.at[j]], ssems[X], add=True)

    pltpu.make_async_copy(
        bufs[2], acc.at[tgt_v.at[CHUNKS_PER_W - 1]], ssems[2]).wait()

    plsc.subcore_barrier()
    for b in range(5):
        rbase = s * 625 + b * 125
        pltpu.sync_copy(acc.at[pl.ds(rbase, 125)], rows0.at[pl.ds(0, 125)])

        @pl.loop(0, 125)
        def _relu(i):
            for h in range(DH // 16):
                fsl = pl.ds(h * 16, 16)
                rows0[i, fsl] = jnp.maximum(rows0[i, fsl], 0.0)

        pltpu.sync_copy(rows0.at[pl.ds(0, 125)],
                        out_hbm.at[pl.ds(rbase, 125), pl.ds(c * DH, DH)])


@jax.jit
def _run(x, edge_index, edge_type, rel_emb, W_lin, W_attn):
    src = edge_index[0].astype(jnp.int32)
    tgt = edge_index[1].astype(jnp.int32)
    typ = edge_type.astype(jnp.int32)

    pad = E_PAD - N_EDGES
    src = jnp.concatenate([src, jnp.full((pad,), N_NODES, jnp.int32)])
    tgt = jnp.concatenate([tgt, jnp.zeros((pad,), jnp.int32)])
    typ = jnp.concatenate([typ, jnp.zeros((pad,), jnp.int32)])
    src = src.reshape(NS, CHUNKS_PER_W, CHUNK)
    tgt = tgt.reshape(NS, CHUNKS_PER_W, CHUNK)
    typ = typ.reshape(NS, CHUNKS_PER_W, CHUNK)
    src_off = src[None] + (jnp.arange(NC, dtype=jnp.int32) * N_PAD)[
        :, None, None, None]

    rel_pad = jnp.concatenate(
        [rel_emb, jnp.zeros((64 - N_REL, D), jnp.float32)], axis=0)

    xt_split, s_pad, r_pad = pl.pallas_call(
        _tc_prep,
        out_shape=[
            jax.ShapeDtypeStruct((NC * N_PAD, DH), jnp.float32),
            jax.ShapeDtypeStruct((N_PAD, 1), jnp.float32),
            jax.ShapeDtypeStruct((64, 1), jnp.float32),
        ],
    )(x, W_lin, W_attn, rel_pad)

    s1 = s_pad.reshape(N_PAD)
    r1 = r_pad.reshape(64)

    mesh = plsc.VectorSubcoreMesh(
        core_axis_name="c", subcore_axis_name="s",
        num_cores=NC, num_subcores=NS)
    sc_call = pl.kernel(
        _sc_edges,
        out_type=jax.ShapeDtypeStruct((N_NODES, D), jnp.float32),
        mesh=mesh,
        compiler_params=pltpu.CompilerParams(
            needs_layout_passes=False, use_tc_tiling_on_sc=False),
        scratch_types=[
            pltpu.VMEM((CHUNKS_PER_W, CHUNK), jnp.int32),
            pltpu.VMEM((CHUNKS_PER_W, CHUNK), jnp.int32),
            pltpu.VMEM((N_PAD,), jnp.float32),
            pltpu.VMEM((64,), jnp.float32),
            pltpu.VMEM((3, CHUNK), jnp.int32),
            pltpu.VMEM((CHUNK, DH), jnp.float32),
            pltpu.VMEM((CHUNK, DH), jnp.float32),
            pltpu.VMEM((CHUNK, DH), jnp.float32),
            pltpu.VMEM_SHARED((ACC_ROWS, DH), jnp.float32),
            pltpu.SemaphoreType.DMA,
            pltpu.SemaphoreType.DMA,
            pltpu.SemaphoreType.DMA,
            pltpu.SemaphoreType.DMA,
            pltpu.SemaphoreType.DMA,
            pltpu.SemaphoreType.DMA,
            pltpu.SemaphoreType.DMA,
            pltpu.SemaphoreType.DMA,
            pltpu.SemaphoreType.DMA,
            pltpu.SemaphoreType.DMA,
            pltpu.SemaphoreType.DMA,
            pltpu.SemaphoreType.DMA,
        ],
    )
    out = sc_call(xt_split, s1, r1, src_off, tgt, typ)
    return out


def kernel(x, edge_index, edge_type, rel_emb, W_lin, W_attn):
    return _run(x, edge_index, edge_type, rel_emb, W_lin, W_attn)


# speedup vs baseline: 1.6825x; 1.0037x over previous
import jax
import jax.numpy as jnp
from jax import lax
from jax.experimental import pallas as pl
from jax.experimental.pallas import tpu as pltpu
from jax.experimental.pallas import tpu_sc as plsc

N_NODES = 10000
N_EDGES = 320000
D = 128
DH = D // 2
N_REL = 50

NC = 2
NS = 16
CHUNK = 128
CHUNKS_PER_W = 159
EPW = CHUNKS_PER_W * CHUNK
E_PAD = NS * EPW
N_PAD = 10016
ACC_ROWS = 10240
ROWS_PER_TILE = ACC_ROWS // NS


def _tc_prep(x_ref, wl_ref, wa_ref, rel_ref, xt_ref, s_ref, r_ref):
    xv = x_ref[...]
    xt = lax.dot_general(
        xv, wl_ref[...], (((1,), (1,)), ((), ())),
        preferred_element_type=jnp.float32)
    pad_z = jnp.zeros((N_PAD - N_NODES, DH), jnp.float32)
    xt_ref[0:N_NODES, :] = xt[:, 0:DH]
    xt_ref[N_NODES:N_PAD, :] = pad_z
    xt_ref[N_PAD:N_PAD + N_NODES, :] = xt[:, DH:D]
    xt_ref[N_PAD + N_NODES:2 * N_PAD, :] = pad_z
    wa = wa_ref[...]
    sv = lax.dot_general(
        xv, wa, (((1,), (1,)), ((), ())), preferred_element_type=jnp.float32)
    s_ref[0:N_NODES, :] = sv
    s_ref[N_NODES:N_PAD, :] = jnp.zeros((N_PAD - N_NODES, 1), jnp.float32)
    r_ref[...] = lax.dot_general(
        rel_ref[...], wa, (((1,), (1,)), ((), ())),
        preferred_element_type=jnp.float32)


def _sc_edges(xt_hbm, s_hbm, r_hbm, src_hbm, tgt_hbm, typ_hbm, out_hbm,
              src_v, tgt_v, s_v, r_v, typb,
              rows0, rows1, rows2, acc,
              gsem0, gsem1, gsem2, ssem0, ssem1, ssem2,
              tsem0, tsem1, tsem2, gsem0b, gsem1b, gsem2b):
    c = lax.axis_index("c")
    s = lax.axis_index("s")
    bufs = (rows0, rows1, rows2)
    gsems = (gsem0, gsem1, gsem2)
    gsembs = (gsem0b, gsem1b, gsem2b)
    ssems = (ssem0, ssem1, ssem2)
    tsems = (tsem0, tsem1, tsem2)

    @pl.loop(0, CHUNK)
    def _zero_rows(i):
        zero16 = jnp.zeros((16,), jnp.float32)
        for h in range(DH // 16):
            rows0[i, pl.ds(h * 16, 16)] = zero16

    for b in range(ROWS_PER_TILE // CHUNK):
        pltpu.sync_copy(rows0, acc.at[pl.ds((s * 5 + b) * CHUNK, CHUNK)])
    plsc.subcore_barrier()

    pltpu.sync_copy(src_hbm.at[c, s], src_v)
    pltpu.sync_copy(tgt_hbm.at[s], tgt_v)
    pltpu.sync_copy(s_hbm, s_v)
    pltpu.sync_copy(r_hbm, r_v)

    coff = c * N_PAD

    def _scale(j, rows_x, X):
        @plsc.parallel_loop(0, CHUNK // 16, unroll=4)
        def _grp(k):
            sl = pl.ds(k * 16, 16)
            idx16 = src_v[j, sl] - coff
            typ16 = typb[X, sl]
            sv = plsc.load_gather(s_v, [idx16])
            rv = plsc.load_gather(r_v, [typ16])
            a16 = 1.0 / (1.0 + jnp.exp(-(sv + rv)))
            base = k * 16
            for l in range(16):
                a = lax.broadcast_in_dim(a16[l], (16,), ())
                for h in range(DH // 16):
                    fsl = pl.ds(h * 16, 16)
                    rows_x[base + l, fsl] = rows_x[base + l, fsl] * a

    HB = CHUNK // 2

    def _gth(j, X):
        pltpu.async_copy(xt_hbm.at[src_v.at[j, pl.ds(0, HB)]],
                         bufs[X].at[pl.ds(0, HB)], gsems[X])
        pltpu.async_copy(xt_hbm.at[src_v.at[j, pl.ds(HB, HB)]],
                         bufs[X].at[pl.ds(HB, HB)], gsembs[X])

    def _gth_wait(j, X):
        pltpu.make_async_copy(xt_hbm.at[src_v.at[j, pl.ds(0, HB)]],
                              bufs[X].at[pl.ds(0, HB)], gsems[X]).wait()
        pltpu.make_async_copy(xt_hbm.at[src_v.at[j, pl.ds(HB, HB)]],
                              bufs[X].at[pl.ds(HB, HB)], gsembs[X]).wait()

    _gth(0, 0)
    _gth(1, 1)
    pltpu.async_copy(typ_hbm.at[s, 0], typb.at[0], tsem0)
    pltpu.async_copy(typ_hbm.at[s, 1], typb.at[1], tsem1)

    @pl.loop(0, CHUNKS_PER_W, step=3)
    def _t(t):
        for i in range(3):
            j = t + i
            X = i
            Z = (i + 2) % 3
            _gth_wait(j, X)
            pltpu.make_async_copy(
                typ_hbm.at[s, j], typb.at[X], tsems[X]).wait()
            _scale(j, bufs[X], X)
            if i == 0:
                @pl.when(t >= 1)
                def _():
                    pltpu.make_async_copy(
                        bufs[Z], acc.at[tgt_v.at[j - 1]], ssems[Z]).wait()
                _gth(j + 2, Z)
                pltpu.async_copy(typ_hbm.at[s, j + 2], typb.at[Z], tsems[Z])
            else:
                pltpu.make_async_copy(
                    bufs[Z], acc.at[tgt_v.at[j - 1]], ssems[Z]).wait()

                @pl.when(j + 2 < CHUNKS_PER_W)
                def _():
                    _gth(j + 2, Z)
                    pltpu.async_copy(
                        typ_hbm.at[s, j + 2], typb.at[Z], tsems[Z])
            pltpu.async_copy(bufs[X], acc.at[tgt_v.at[j]], ssems[X], add=True)

    pltpu.make_async_copy(
        bufs[2], acc.at[tgt_v.at[CHUNKS_PER_W - 1]], ssems[2]).wait()

    plsc.subcore_barrier()
    for b in range(5):
        rbase = s * 625 + b * 125
        pltpu.sync_copy(acc.at[pl.ds(rbase, 125)], rows0.at[pl.ds(0, 125)])

        @pl.loop(0, 125)
        def _relu(i):
            for h in range(DH // 16):
                fsl = pl.ds(h * 16, 16)
                rows0[i, fsl] = jnp.maximum(rows0[i, fsl], 0.0)

        pltpu.sync_copy(rows0.at[pl.ds(0, 125)],
                        out_hbm.at[pl.ds(rbase, 125), pl.ds(c * DH, DH)])


@jax.jit
def _run(x, edge_index, edge_type, rel_emb, W_lin, W_attn):
    src = edge_index[0].astype(jnp.int32)
    tgt = edge_index[1].astype(jnp.int32)
    typ = edge_type.astype(jnp.int32)

    pad = E_PAD - N_EDGES
    src = jnp.concatenate([src, jnp.full((pad,), N_NODES, jnp.int32)])
    tgt = jnp.concatenate([tgt, jnp.zeros((pad,), jnp.int32)])
    typ = jnp.concatenate([typ, jnp.zeros((pad,), jnp.int32)])
    src = src.reshape(NS, CHUNKS_PER_W, CHUNK)
    tgt = tgt.reshape(NS, CHUNKS_PER_W, CHUNK)
    typ = typ.reshape(NS, CHUNKS_PER_W, CHUNK)
    src_off = src[None] + (jnp.arange(NC, dtype=jnp.int32) * N_PAD)[
        :, None, None, None]

    rel_pad = jnp.concatenate(
        [rel_emb, jnp.zeros((64 - N_REL, D), jnp.float32)], axis=0)

    xt_split, s_pad, r_pad = pl.pallas_call(
        _tc_prep,
        out_shape=[
            jax.ShapeDtypeStruct((NC * N_PAD, DH), jnp.float32),
            jax.ShapeDtypeStruct((N_PAD, 1), jnp.float32),
            jax.ShapeDtypeStruct((64, 1), jnp.float32),
        ],
    )(x, W_lin, W_attn, rel_pad)

    s1 = s_pad.reshape(N_PAD)
    r1 = r_pad.reshape(64)

    mesh = plsc.VectorSubcoreMesh(
        core_axis_name="c", subcore_axis_name="s",
        num_cores=NC, num_subcores=NS)
    sc_call = pl.kernel(
        _sc_edges,
        out_type=jax.ShapeDtypeStruct((N_NODES, D), jnp.float32),
        mesh=mesh,
        compiler_params=pltpu.CompilerParams(
            needs_layout_passes=False, use_tc_tiling_on_sc=False),
        scratch_types=[
            pltpu.VMEM((CHUNKS_PER_W, CHUNK), jnp.int32),
            pltpu.VMEM((CHUNKS_PER_W, CHUNK), jnp.int32),
            pltpu.VMEM((N_PAD,), jnp.float32),
            pltpu.VMEM((64,), jnp.float32),
            pltpu.VMEM((3, CHUNK), jnp.int32),
            pltpu.VMEM((CHUNK, DH), jnp.float32),
            pltpu.VMEM((CHUNK, DH), jnp.float32),
            pltpu.VMEM((CHUNK, DH), jnp.float32),
            pltpu.VMEM_SHARED((ACC_ROWS, DH), jnp.float32),
            pltpu.SemaphoreType.DMA,
            pltpu.SemaphoreType.DMA,
            pltpu.SemaphoreType.DMA,
            pltpu.SemaphoreType.DMA,
            pltpu.SemaphoreType.DMA,
            pltpu.SemaphoreType.DMA,
            pltpu.SemaphoreType.DMA,
            pltpu.SemaphoreType.DMA,
            pltpu.SemaphoreType.DMA,
            pltpu.SemaphoreType.DMA,
            pltpu.SemaphoreType.DMA,
            pltpu.SemaphoreType.DMA,
        ],
    )
    out = sc_call(xt_split, s1, r1, src_off, tgt, typ)
    return out


def kernel(x, edge_index, edge_type, rel_emb, W_lin, W_attn):
    return _run(x, edge_index, edge_type, rel_emb, W_lin, W_attn)
